# baseline (speedup 1.0000x reference)
import functools

import jax
import jax.numpy as jnp
from jax import lax
from jax.experimental import pallas as pl
from jax.experimental.pallas import tpu as pltpu
import jax.experimental.pallas.tpu_sc as plsc

N_P = 50000
N_A = 20000
D = 128
NC = 2
NS = 16
LANES = 16
B = 96
BD = 128
GR = NS * BD

CH_P = 12544
ACC_P = 12800
NPASS_P = 2
CH_A = 10000
ACC_A = 10368
NPASS_A = 1


def _round_up(x, m):
    return -(-x // m) * m

_MESH = plsc.VectorSubcoreMesh(core_axis_name="c", subcore_axis_name="s",
                               num_cores=NC, num_subcores=NS)


def _chunks(total, step):
    out = []
    r = 0
    while r < total:
        out.append((r, min(step, total - r)))
        r += step
    return out


def _make_agg3(cfgs):
    acc_max = max(acc_rows for (_, _, acc_rows, _) in cfgs)

    def body(*refs):
        xc, xw, xr, sch, dch, swh, dwh, srh, drh = refs[:9]
        outs = refs[9:12]
        (src0, src1, dst0, dst1, dl0, dl1, rows0, rows1,
         zbuf, acc, gs0, gs1, is0, is1) = refs[12:]
        xs_hbm = (xc, xw, xr)
        srcs_hbm = (sch, swh, srh)
        dsts_hbm = (dch, dwh, drh)
        c = lax.axis_index("c")
        s = lax.axis_index("s")
        iota = lax.iota(jnp.int32, LANES)
        srcs = (src0, src1)
        dsts = (dst0, dst1)
        dls = (dl0, dl1)
        rows = (rows0, rows1)
        gss = (gs0, gs1)
        iss = (is0, is1)

        def zb_fill(i, _):
            for j in range(D // LANES):
                zbuf[i, pl.ds(j * LANES, LANES)] = jnp.zeros((LANES,), jnp.float32)
            return 0
        lax.fori_loop(0, 16, zb_fill, 0)

        for r, (nb, ch, acc_rows, npass) in enumerate(cfgs):
            et = nb * B
            rpt = acc_rows // NS
            zw = _chunks(rpt, 16)
            x_hbm = xs_hbm[r]
            src_hbm = srcs_hbm[r]
            dst_hbm = dsts_hbm[r]
            out_hbm = outs[r]
            trash = ch + s * LANES + iota
            base = s * et

            def fetch_idx(buf, i):
                off = base + i * B
                pltpu.async_copy(src_hbm.at[pl.ds(off, B)], srcs[buf], iss[buf])
                pltpu.async_copy(dst_hbm.at[pl.ds(off, B)], dsts[buf], iss[buf])

            def wait_idx(buf):
                pltpu.make_async_copy(src_hbm.at[pl.ds(0, B)], srcs[buf],
                                      iss[buf]).wait()
                pltpu.make_async_copy(dst_hbm.at[pl.ds(0, B)], dsts[buf],
                                      iss[buf]).wait()

            def issue_gather(buf):
                pltpu.async_copy(x_hbm.at[srcs[buf]], rows[buf], gss[buf])

            def wait_gather(buf):
                pltpu.make_async_copy(x_hbm.at[srcs[buf]], rows[buf],
                                      gss[buf]).wait()

            def compute_dloc(buf, lo, hi):
                for j in range(B // LANES):
                    dvec = dsts[buf][pl.ds(j * LANES, LANES)]
                    inr = (dvec >= lo) & (dvec < hi)
                    dls[buf][pl.ds(j * LANES, LANES)] = jnp.where(
                        inr, dvec - lo, trash)

            def scatter(buf):
                pltpu.sync_copy(rows[buf], acc.at[dls[buf]], add=True)

            for p in range(npass):
                chunk = c * npass + p
                lo = chunk * ch
                hi = lo + ch

                for (r0, rn) in zw:
                    pltpu.sync_copy(zbuf.at[pl.ds(0, rn)],
                                    acc.at[pl.ds(s * rpt + r0, rn)])
                plsc.subcore_barrier()

                pltpu.sync_copy(src_hbm.at[pl.ds(base, B)], src0)
                pltpu.sync_copy(dst_hbm.at[pl.ds(base, B)], dst0)
                compute_dloc(0, lo, hi)
                issue_gather(0)
                fetch_idx(1, 1)

                def grp(g, _):
                    for sub in range(2):
                        i = 2 * g + sub
                        cur = sub
                        nxt = 1 - sub
                        wait_idx(nxt)
                        compute_dloc(nxt, lo, hi)
                        issue_gather(nxt)
                        wait_gather(cur)

                        @pl.when(i + 2 < nb)
                        def _():
                            fetch_idx(cur, i + 2)
                        scatter(cur)
                    return 0
                lax.fori_loop(0, (nb - 1) // 2, grp, 0)

                wait_gather(0)
                scatter(0)
                plsc.subcore_barrier()

                pltpu.sync_copy(acc.at[pl.ds(s * rpt, rpt)],
                                out_hbm.at[pl.ds(chunk * acc_rows + s * rpt, rpt)])
                plsc.subcore_barrier()

    kern = pl.kernel(
        body,
        out_type=[jax.ShapeDtypeStruct((NC * npass * acc_rows, D), jnp.float32)
                  for (_, _, acc_rows, npass) in cfgs],
        mesh=_MESH,
        scratch_types=[
            pltpu.VMEM((B,), jnp.int32),
            pltpu.VMEM((B,), jnp.int32),
            pltpu.VMEM((B,), jnp.int32),
            pltpu.VMEM((B,), jnp.int32),
            pltpu.VMEM((B,), jnp.int32),
            pltpu.VMEM((B,), jnp.int32),
            pltpu.VMEM((B, D), jnp.float32),
            pltpu.VMEM((B, D), jnp.float32),
            pltpu.VMEM((16, D), jnp.float32),
            pltpu.VMEM_SHARED((acc_max, D), jnp.float32),
            pltpu.SemaphoreType.DMA,
            pltpu.SemaphoreType.DMA,
            pltpu.SemaphoreType.DMA,
            pltpu.SemaphoreType.DMA,
        ],
        compiler_params=pltpu.CompilerParams(use_tc_tiling_on_sc=False),
    )
    return kern


def _unpad_agg(out, npass, acc_rows, ch, n):
    return out.reshape(NC * npass, acc_rows, D)[:, :ch].reshape(-1, D)[:n]


def _make_deg(e_pads, n_nodes, owners):
    n_arr = len(e_pads)
    n_hist = [_round_up(n + LANES, 8 * NS) for n in n_nodes]
    acc_rows = max(n_hist)

    def body(*refs):
        ins = refs[:n_arr]
        outs = refs[n_arr:2 * n_arr]
        idx0, idx1, onesb, zb, acc, is0, is1 = refs[2 * n_arr:]
        c = lax.axis_index("c")
        s = lax.axis_index("s")
        idxs = (idx0, idx1)
        iss = (is0, is1)

        def zb_fill(i, _):
            zb[i, pl.ds(0, LANES)] = jnp.zeros((LANES,), jnp.float32)
            return 0
        lax.fori_loop(0, 1024, zb_fill, 0)

        def ob_fill(i, _):
            onesb[i, pl.ds(0, LANES)] = jnp.ones((LANES,), jnp.float32)
            return 0
        lax.fori_loop(0, BD, ob_fill, 0)

        for a in range(n_arr):
            nh = n_hist[a]
            et = e_pads[a] // NS
            nb = et // BD
            rpt = nh // NS
            idx_hbm = ins[a]
            out_hbm = outs[a]
            base = s * et

            @pl.when(c == owners[a])
            def _():
                for (r0, rn) in _chunks(rpt, 1024):
                    pltpu.sync_copy(zb.at[pl.ds(0, rn)],
                                    acc.at[pl.ds(s * rpt + r0, rn)])
                plsc.subcore_barrier()

                pltpu.sync_copy(idx_hbm.at[pl.ds(base, BD)], idx0)
                pltpu.async_copy(idx_hbm.at[pl.ds(base + BD, BD)], idx1, is1)

                def cnt(g, _):
                    for sub in range(2):
                        i = 2 * g + sub
                        cur = sub
                        nxt = 1 - sub
                        pltpu.sync_copy(onesb, acc.at[idxs[cur]], add=True)

                        @pl.when(i + 2 < nb)
                        def _():
                            pltpu.async_copy(
                                idx_hbm.at[pl.ds(base + (i + 2) * BD, BD)],
                                idxs[cur], iss[cur])
                        pltpu.make_async_copy(
                            idx_hbm.at[pl.ds(0, BD)], idxs[nxt], iss[nxt]).wait()
                    return 0
                lax.fori_loop(0, (nb - 1) // 2, cnt, 0)
                pltpu.sync_copy(onesb, acc.at[idxs[0]], add=True)
                plsc.subcore_barrier()

                pltpu.sync_copy(acc.at[pl.ds(s * rpt, rpt)],
                                out_hbm.at[pl.ds(s * rpt, rpt)])
                plsc.subcore_barrier()

    kern = pl.kernel(
        body,
        out_type=[jax.ShapeDtypeStruct((n, LANES), jnp.float32)
                  for n in n_hist],
        mesh=_MESH,
        scratch_types=[
            pltpu.VMEM((BD,), jnp.int32),
            pltpu.VMEM((BD,), jnp.int32),
            pltpu.VMEM((BD, LANES), jnp.float32),
            pltpu.VMEM((1024, LANES), jnp.float32),
            pltpu.VMEM_SHARED((acc_rows, LANES), jnp.float32),
            pltpu.SemaphoreType.DMA,
            pltpu.SemaphoreType.DMA,
        ],
        compiler_params=pltpu.CompilerParams(use_tc_tiling_on_sc=False),
    )
    return kern


BR = 2000


def _rs(h):
    return lax.rsqrt(jnp.maximum(h[:, 0:1], 1.0))


def _scale2_body(x_ref, h1_ref, h2_ref, o1_ref, o2_ref):
    x = x_ref[...]
    o1_ref[...] = x * _rs(h1_ref[...])
    o2_ref[...] = x * _rs(h2_ref[...])


def _scale1_body(x_ref, h1_ref, o1_ref):
    o1_ref[...] = x_ref[...] * _rs(h1_ref[...])


def _post2s_body(a1_ref, a2_ref, w1_ref, w2_ref, hd1_ref, hd2_ref,
                 hs1_ref, hs2_ref, b_ref, o1_ref, o2_ref):
    h = (_rs(hd1_ref[...]) * jnp.dot(a1_ref[...], w1_ref[...],
                                     preferred_element_type=jnp.float32)
         + _rs(hd2_ref[...]) * jnp.dot(a2_ref[...], w2_ref[...],
                                       preferred_element_type=jnp.float32)
         + b_ref[...])
    o1_ref[...] = h * _rs(hs1_ref[...])
    o2_ref[...] = h * _rs(hs2_ref[...])


def _post2f_body(a1_ref, a2_ref, w1_ref, w2_ref, hd1_ref, hd2_ref, b_ref, o_ref):
    o_ref[...] = (_rs(hd1_ref[...]) * jnp.dot(a1_ref[...], w1_ref[...],
                                              preferred_element_type=jnp.float32)
                  + _rs(hd2_ref[...]) * jnp.dot(a2_ref[...], w2_ref[...],
                                                preferred_element_type=jnp.float32)
                  + b_ref[...])


def _post1s_body(a1_ref, w1_ref, hd1_ref, hs1_ref, b_ref, o_ref):
    h = (_rs(hd1_ref[...]) * jnp.dot(a1_ref[...], w1_ref[...],
                                     preferred_element_type=jnp.float32)
         + b_ref[...])
    o_ref[...] = h * _rs(hs1_ref[...])


def _post1f_body(a1_ref, w1_ref, hd1_ref, b_ref, o_ref):
    o_ref[...] = (_rs(hd1_ref[...]) * jnp.dot(a1_ref[...], w1_ref[...],
                                              preferred_element_type=jnp.float32)
                  + b_ref[...])


def _x_spec():
    return pl.BlockSpec((BR, D), lambda i: (i, 0))


def _h_spec():
    return pl.BlockSpec((BR, 1), lambda i: (i, 0))


def _w_spec():
    return pl.BlockSpec((D, D), lambda i: (0, 0))


def _b_spec():
    return pl.BlockSpec((1, D), lambda i: (0, 0))


def _tc_call(body, n, in_specs, n_out):
    out_spec = _x_spec()
    shape = jax.ShapeDtypeStruct((n, D), jnp.float32)
    if n_out == 1:
        return pl.pallas_call(body, grid=(n // BR,), in_specs=in_specs,
                              out_specs=out_spec, out_shape=shape)
    return pl.pallas_call(body, grid=(n // BR,), in_specs=in_specs,
                          out_specs=[out_spec] * n_out,
                          out_shape=[shape] * n_out)


def _pad_len(e):
    nb = -(-e // GR)
    return (nb + 1 - (nb % 2)) * GR


def _agg_nb(e):
    nb = -(-e // (NS * B))
    return nb + 1 - (nb % 2)


def _pad_agg_edges(src, dst, n_src):
    e = src.shape[0]
    p = _agg_nb(e) * NS * B - e
    ar = jnp.arange(p, dtype=jnp.int32)
    src_p = jnp.concatenate([src, (ar * 911) % n_src])
    dst_p = jnp.concatenate([dst, jnp.full((p,), -1, jnp.int32)])
    return src_p, dst_p


def _pad_deg_edges(idx, n):
    e = idx.shape[0]
    p = _pad_len(e) - e
    ar = jnp.arange(p, dtype=jnp.int32)
    return jnp.concatenate([idx, n + (ar % LANES)])


def kernel(xs, cites_src, cites_dst, writes_src, writes_dst, rev_writes_src,
           rev_writes_dst, emb_author, W1_cites, b1_cites, W1_writes, b1_writes,
           W1_rev, b1_rev, W2_cites, b2_cites, W2_writes, b2_writes, W2_rev,
           b2_rev):
    ec = _pad_len(cites_src.shape[0])
    ew = _pad_len(writes_src.shape[0])
    er = _pad_len(rev_writes_src.shape[0])
    nb_c = _agg_nb(cites_src.shape[0])
    nb_w = _agg_nb(writes_src.shape[0])
    nb_r = _agg_nb(rev_writes_src.shape[0])

    cs_a, cd_a = _pad_agg_edges(cites_src, cites_dst, N_P)
    ws_a, wd_a = _pad_agg_edges(writes_src, writes_dst, N_A)
    rs_a, rd_a = _pad_agg_edges(rev_writes_src, rev_writes_dst, N_P)

    deg_in = [_pad_deg_edges(cites_src, N_P), _pad_deg_edges(cites_dst, N_P),
              _pad_deg_edges(writes_src, N_A), _pad_deg_edges(writes_dst, N_P),
              _pad_deg_edges(rev_writes_src, N_P), _pad_deg_edges(rev_writes_dst, N_A)]

    deg_kern = _make_deg([ec, ec, ew, ew, er, er],
                         [N_P, N_P, N_A, N_P, N_P, N_A],
                         [0, 0, 1, 1, 1, 1])
    hists = deg_kern(*deg_in)
    hs_c = hists[0][:N_P, 0:1]
    hd_c = hists[1][:N_P, 0:1]
    hs_w = hists[2][:N_A, 0:1]
    hd_w = hists[3][:N_P, 0:1]
    hs_r = hists[4][:N_P, 0:1]
    hd_r = hists[5][:N_A, 0:1]

    agg3_k = _make_agg3([(nb_c, CH_P, ACC_P, NPASS_P),
                         (nb_w, CH_P, ACC_P, NPASS_P),
                         (nb_r, CH_A, ACC_A, NPASS_A)])

    def agg_layer(x_c, x_w, x_r):
        raw_c, raw_w, raw_r = agg3_k(x_c, x_w, x_r,
                                     cs_a, cd_a, ws_a, wd_a, rs_a, rd_a)
        return (_unpad_agg(raw_c, NPASS_P, ACC_P, CH_P, N_P),
                _unpad_agg(raw_w, NPASS_P, ACC_P, CH_P, N_P),
                _unpad_agg(raw_r, NPASS_A, ACC_A, CH_A, N_A))

    xn_c, xn_r = _tc_call(_scale2_body, N_P,
                          [_x_spec(), _h_spec(), _h_spec()], 2)(xs, hs_c, hs_r)
    xn_w = _tc_call(_scale1_body, N_A,
                    [_x_spec(), _h_spec()], 1)(emb_author, hs_w)

    agg_c, agg_w, agg_r = agg_layer(xn_c, xn_w, xn_r)

    post2s = _tc_call(_post2s_body, N_P,
                      [_x_spec(), _x_spec(), _w_spec(), _w_spec(), _h_spec(),
                       _h_spec(), _h_spec(), _h_spec(), _b_spec()], 2)
    hn_c, hn_r = post2s(agg_c, agg_w, W1_cites, W1_writes, hd_c, hd_w,
                        hs_c, hs_r, (b1_cites + b1_writes).reshape(1, D))
    post1s = _tc_call(_post1s_body, N_A,
                      [_x_spec(), _w_spec(), _h_spec(), _h_spec(), _b_spec()], 1)
    hn_w = post1s(agg_r, W1_rev, hd_r, hs_w, b1_rev.reshape(1, D))

    agg2_c, agg2_w, agg2_r = agg_layer(hn_c, hn_w, hn_r)

    post2f = _tc_call(_post2f_body, N_P,
                      [_x_spec(), _x_spec(), _w_spec(), _w_spec(), _h_spec(),
                       _h_spec(), _b_spec()], 1)
    out_paper = post2f(agg2_c, agg2_w, W2_cites, W2_writes, hd_c, hd_w,
                       (b2_cites + b2_writes).reshape(1, D))
    post1f = _tc_call(_post1f_body, N_A,
                      [_x_spec(), _w_spec(), _h_spec(), _b_spec()], 1)
    out_author = post1f(agg2_r, W2_rev, hd_r, b2_rev.reshape(1, D))

    return (out_paper, out_author)


# speedup vs baseline: 1.0139x; 1.0139x over previous
import functools

import jax
import jax.numpy as jnp
from jax import lax
from jax.experimental import pallas as pl
from jax.experimental.pallas import tpu as pltpu
import jax.experimental.pallas.tpu_sc as plsc

N_P = 50000
N_A = 20000
D = 128
NC = 2
NS = 16
LANES = 16
B = 96
BD = 128
GR = NS * BD

CH_P = 12544
ACC_P = 12800
NPASS_P = 2
CH_A = 10000
ACC_A = 10368
NPASS_A = 1


def _round_up(x, m):
    return -(-x // m) * m

_MESH = plsc.VectorSubcoreMesh(core_axis_name="c", subcore_axis_name="s",
                               num_cores=NC, num_subcores=NS)


def _chunks(total, step):
    out = []
    r = 0
    while r < total:
        out.append((r, min(step, total - r)))
        r += step
    return out


def _make_agg(nb, ch, acc_rows, npass):
    et = nb * B
    out_rows = NC * npass * acc_rows
    rpt = acc_rows // NS
    zw = _chunks(rpt, 16)

    def body(x_hbm, src_hbm, dst_hbm, tok_hbm, out_hbm,
             src0, src1, dst0, dst1, dl0, dl1, rows0, rows1,
             zbuf, acc, gs0, gs1, is0, is1):
        c = lax.axis_index("c")
        s = lax.axis_index("s")
        iota = lax.iota(jnp.int32, LANES)
        srcs = (src0, src1)
        dsts = (dst0, dst1)
        dls = (dl0, dl1)
        rows = (rows0, rows1)
        gss = (gs0, gs1)
        iss = (is0, is1)

        def zb_fill(i, _):
            for j in range(D // LANES):
                zbuf[i, pl.ds(j * LANES, LANES)] = jnp.zeros((LANES,), jnp.float32)
            return 0
        lax.fori_loop(0, 16, zb_fill, 0)

        trash = ch + s * LANES + iota
        base = s * et

        def fetch_idx(buf, i):
            off = base + i * B
            pltpu.async_copy(src_hbm.at[pl.ds(off, B)], srcs[buf], iss[buf])
            pltpu.async_copy(dst_hbm.at[pl.ds(off, B)], dsts[buf], iss[buf])

        def wait_idx(buf):
            pltpu.make_async_copy(src_hbm.at[pl.ds(0, B)], srcs[buf], iss[buf]).wait()
            pltpu.make_async_copy(dst_hbm.at[pl.ds(0, B)], dsts[buf], iss[buf]).wait()

        def issue_gather(buf):
            pltpu.async_copy(x_hbm.at[srcs[buf]], rows[buf], gss[buf])

        def wait_gather(buf):
            pltpu.make_async_copy(x_hbm.at[srcs[buf]], rows[buf], gss[buf]).wait()

        def compute_dloc(buf, lo, hi):
            for j in range(B // LANES):
                dvec = dsts[buf][pl.ds(j * LANES, LANES)]
                inr = (dvec >= lo) & (dvec < hi)
                dls[buf][pl.ds(j * LANES, LANES)] = jnp.where(inr, dvec - lo, trash)

        def scatter(buf):
            pltpu.sync_copy(rows[buf], acc.at[dls[buf]], add=True)

        for p in range(npass):
            chunk = c * npass + p
            lo = chunk * ch
            hi = lo + ch

            for (r0, rn) in zw:
                pltpu.sync_copy(zbuf.at[pl.ds(0, rn)],
                                acc.at[pl.ds(s * rpt + r0, rn)])
            plsc.subcore_barrier()

            pltpu.sync_copy(src_hbm.at[pl.ds(base, B)], src0)
            pltpu.sync_copy(dst_hbm.at[pl.ds(base, B)], dst0)
            compute_dloc(0, lo, hi)
            issue_gather(0)
            fetch_idx(1, 1)

            def grp(g, _):
                for sub in range(2):
                    i = 2 * g + sub
                    cur = sub
                    nxt = 1 - sub
                    wait_idx(nxt)
                    compute_dloc(nxt, lo, hi)
                    issue_gather(nxt)
                    wait_gather(cur)

                    @pl.when(i + 2 < nb)
                    def _():
                        fetch_idx(cur, i + 2)
                    scatter(cur)
                return 0
            lax.fori_loop(0, (nb - 1) // 2, grp, 0)

            wait_gather(0)
            scatter(0)
            plsc.subcore_barrier()

            pltpu.sync_copy(acc.at[pl.ds(s * rpt, rpt)],
                            out_hbm.at[pl.ds(chunk * acc_rows + s * rpt, rpt)])
            plsc.subcore_barrier()

    kern = pl.kernel(
        body,
        out_type=jax.ShapeDtypeStruct((out_rows, D), jnp.float32),
        mesh=_MESH,
        scratch_types=[
            pltpu.VMEM((B,), jnp.int32),
            pltpu.VMEM((B,), jnp.int32),
            pltpu.VMEM((B,), jnp.int32),
            pltpu.VMEM((B,), jnp.int32),
            pltpu.VMEM((B,), jnp.int32),
            pltpu.VMEM((B,), jnp.int32),
            pltpu.VMEM((B, D), jnp.float32),
            pltpu.VMEM((B, D), jnp.float32),
            pltpu.VMEM((16, D), jnp.float32),
            pltpu.VMEM_SHARED((acc_rows, D), jnp.float32),
            pltpu.SemaphoreType.DMA,
            pltpu.SemaphoreType.DMA,
            pltpu.SemaphoreType.DMA,
            pltpu.SemaphoreType.DMA,
        ],
        compiler_params=pltpu.CompilerParams(use_tc_tiling_on_sc=False),
    )
    return kern


def _unpad_agg(out, npass, acc_rows, ch, n):
    return out.reshape(NC * npass, acc_rows, D)[:, :ch].reshape(-1, D)[:n]


def _make_deg(e_pads, n_nodes, owners):
    n_arr = len(e_pads)
    n_hist = [_round_up(n + LANES, 8 * NS) for n in n_nodes]
    acc_rows = max(n_hist)

    def body(*refs):
        ins = refs[:n_arr]
        outs = refs[n_arr:2 * n_arr]
        idx0, idx1, onesb, zb, acc, is0, is1 = refs[2 * n_arr:]
        c = lax.axis_index("c")
        s = lax.axis_index("s")
        idxs = (idx0, idx1)
        iss = (is0, is1)

        def zb_fill(i, _):
            zb[i, pl.ds(0, LANES)] = jnp.zeros((LANES,), jnp.float32)
            return 0
        lax.fori_loop(0, 1024, zb_fill, 0)

        def ob_fill(i, _):
            onesb[i, pl.ds(0, LANES)] = jnp.ones((LANES,), jnp.float32)
            return 0
        lax.fori_loop(0, BD, ob_fill, 0)

        for a in range(n_arr):
            nh = n_hist[a]
            et = e_pads[a] // NS
            nb = et // BD
            rpt = nh // NS
            idx_hbm = ins[a]
            out_hbm = outs[a]
            base = s * et

            @pl.when(c == owners[a])
            def _():
                for (r0, rn) in _chunks(rpt, 1024):
                    pltpu.sync_copy(zb.at[pl.ds(0, rn)],
                                    acc.at[pl.ds(s * rpt + r0, rn)])
                plsc.subcore_barrier()

                pltpu.sync_copy(idx_hbm.at[pl.ds(base, BD)], idx0)
                pltpu.async_copy(idx_hbm.at[pl.ds(base + BD, BD)], idx1, is1)

                def cnt(g, _):
                    for sub in range(2):
                        i = 2 * g + sub
                        cur = sub
                        nxt = 1 - sub
                        pltpu.sync_copy(onesb, acc.at[idxs[cur]], add=True)

                        @pl.when(i + 2 < nb)
                        def _():
                            pltpu.async_copy(
                                idx_hbm.at[pl.ds(base + (i + 2) * BD, BD)],
                                idxs[cur], iss[cur])
                        pltpu.make_async_copy(
                            idx_hbm.at[pl.ds(0, BD)], idxs[nxt], iss[nxt]).wait()
                    return 0
                lax.fori_loop(0, (nb - 1) // 2, cnt, 0)
                pltpu.sync_copy(onesb, acc.at[idxs[0]], add=True)
                plsc.subcore_barrier()

                pltpu.sync_copy(acc.at[pl.ds(s * rpt, rpt)],
                                out_hbm.at[pl.ds(s * rpt, rpt)])
                plsc.subcore_barrier()

    kern = pl.kernel(
        body,
        out_type=[jax.ShapeDtypeStruct((n, LANES), jnp.float32)
                  for n in n_hist],
        mesh=_MESH,
        scratch_types=[
            pltpu.VMEM((BD,), jnp.int32),
            pltpu.VMEM((BD,), jnp.int32),
            pltpu.VMEM((BD, LANES), jnp.float32),
            pltpu.VMEM((1024, LANES), jnp.float32),
            pltpu.VMEM_SHARED((acc_rows, LANES), jnp.float32),
            pltpu.SemaphoreType.DMA,
            pltpu.SemaphoreType.DMA,
        ],
        compiler_params=pltpu.CompilerParams(use_tc_tiling_on_sc=False),
    )
    return kern


BR = 2000


def _rs(h):
    return lax.rsqrt(jnp.maximum(h[:, 0:1], 1.0))


def _scale2_body(x_ref, h1_ref, h2_ref, o1_ref, o2_ref):
    x = x_ref[...]
    o1_ref[...] = x * _rs(h1_ref[...])
    o2_ref[...] = x * _rs(h2_ref[...])


def _scale1_body(x_ref, h1_ref, o1_ref):
    o1_ref[...] = x_ref[...] * _rs(h1_ref[...])


def _post2s_body(a1_ref, a2_ref, w1_ref, w2_ref, hd1_ref, hd2_ref,
                 hs1_ref, hs2_ref, b_ref, o1_ref, o2_ref):
    h = (_rs(hd1_ref[...]) * jnp.dot(a1_ref[...], w1_ref[...],
                                     preferred_element_type=jnp.float32)
         + _rs(hd2_ref[...]) * jnp.dot(a2_ref[...], w2_ref[...],
                                       preferred_element_type=jnp.float32)
         + b_ref[...])
    o1_ref[...] = h * _rs(hs1_ref[...])
    o2_ref[...] = h * _rs(hs2_ref[...])


def _post2f_body(a1_ref, a2_ref, w1_ref, w2_ref, hd1_ref, hd2_ref, b_ref, o_ref):
    o_ref[...] = (_rs(hd1_ref[...]) * jnp.dot(a1_ref[...], w1_ref[...],
                                              preferred_element_type=jnp.float32)
                  + _rs(hd2_ref[...]) * jnp.dot(a2_ref[...], w2_ref[...],
                                                preferred_element_type=jnp.float32)
                  + b_ref[...])


def _post1s_body(a1_ref, w1_ref, hd1_ref, hs1_ref, b_ref, o_ref):
    h = (_rs(hd1_ref[...]) * jnp.dot(a1_ref[...], w1_ref[...],
                                     preferred_element_type=jnp.float32)
         + b_ref[...])
    o_ref[...] = h * _rs(hs1_ref[...])


def _post1f_body(a1_ref, w1_ref, hd1_ref, b_ref, o_ref):
    o_ref[...] = (_rs(hd1_ref[...]) * jnp.dot(a1_ref[...], w1_ref[...],
                                              preferred_element_type=jnp.float32)
                  + b_ref[...])


def _x_spec():
    return pl.BlockSpec((BR, D), lambda i: (i, 0))


def _h_spec():
    return pl.BlockSpec((BR, 1), lambda i: (i, 0))


def _w_spec():
    return pl.BlockSpec((D, D), lambda i: (0, 0))


def _b_spec():
    return pl.BlockSpec((1, D), lambda i: (0, 0))


def _tc_call(body, n, in_specs, n_out):
    out_spec = _x_spec()
    shape = jax.ShapeDtypeStruct((n, D), jnp.float32)
    if n_out == 1:
        return pl.pallas_call(body, grid=(n // BR,), in_specs=in_specs,
                              out_specs=out_spec, out_shape=shape)
    return pl.pallas_call(body, grid=(n // BR,), in_specs=in_specs,
                          out_specs=[out_spec] * n_out,
                          out_shape=[shape] * n_out)


def _pad_len(e):
    nb = -(-e // GR)
    return (nb + 1 - (nb % 2)) * GR


def _agg_nb(e):
    nb = -(-e // (NS * B))
    return nb + 1 - (nb % 2)


def _pad_agg_edges(src, dst, n_src):
    e = src.shape[0]
    p = _agg_nb(e) * NS * B - e
    ar = jnp.arange(p, dtype=jnp.int32)
    src_p = jnp.concatenate([src, (ar * 911) % n_src])
    dst_p = jnp.concatenate([dst, jnp.full((p,), -1, jnp.int32)])
    return src_p, dst_p


def _pad_deg_edges(idx, n):
    e = idx.shape[0]
    p = _pad_len(e) - e
    ar = jnp.arange(p, dtype=jnp.int32)
    return jnp.concatenate([idx, n + (ar % LANES)])


def kernel(xs, cites_src, cites_dst, writes_src, writes_dst, rev_writes_src,
           rev_writes_dst, emb_author, W1_cites, b1_cites, W1_writes, b1_writes,
           W1_rev, b1_rev, W2_cites, b2_cites, W2_writes, b2_writes, W2_rev,
           b2_rev):
    ec = _pad_len(cites_src.shape[0])
    ew = _pad_len(writes_src.shape[0])
    er = _pad_len(rev_writes_src.shape[0])
    nb_c = _agg_nb(cites_src.shape[0])
    nb_w = _agg_nb(writes_src.shape[0])
    nb_r = _agg_nb(rev_writes_src.shape[0])

    cs_a, cd_a = _pad_agg_edges(cites_src, cites_dst, N_P)
    ws_a, wd_a = _pad_agg_edges(writes_src, writes_dst, N_A)
    rs_a, rd_a = _pad_agg_edges(rev_writes_src, rev_writes_dst, N_P)

    deg_in = [_pad_deg_edges(cites_src, N_P), _pad_deg_edges(cites_dst, N_P),
              _pad_deg_edges(writes_src, N_A), _pad_deg_edges(writes_dst, N_P),
              _pad_deg_edges(rev_writes_src, N_P), _pad_deg_edges(rev_writes_dst, N_A)]

    deg_kern = _make_deg([ec, ec, ew, ew, er, er],
                         [N_P, N_P, N_A, N_P, N_P, N_A],
                         [0, 0, 1, 1, 1, 1])
    hists = deg_kern(*deg_in)
    hs_c = hists[0][:N_P, 0:1]
    hd_c = hists[1][:N_P, 0:1]
    hs_w = hists[2][:N_A, 0:1]
    hd_w = hists[3][:N_P, 0:1]
    hs_r = hists[4][:N_P, 0:1]
    hd_r = hists[5][:N_A, 0:1]

    agg_c_k = _make_agg(nb_c, CH_P, ACC_P, NPASS_P)
    agg_w_k = _make_agg(nb_w, CH_P, ACC_P, NPASS_P)
    agg_r_k = _make_agg(nb_r, CH_A, ACC_A, NPASS_A)

    def agg_paper(kern, x, s, d, tok):
        raw = kern(x, s, d, tok[:8, :D])
        return _unpad_agg(raw, NPASS_P, ACC_P, CH_P, N_P), raw

    def agg_author(kern, x, s, d, tok):
        raw = kern(x, s, d, tok[:8, :D])
        return _unpad_agg(raw, NPASS_A, ACC_A, CH_A, N_A), raw

    xn_c, xn_r = _tc_call(_scale2_body, N_P,
                          [_x_spec(), _h_spec(), _h_spec()], 2)(xs, hs_c, hs_r)
    xn_w = _tc_call(_scale1_body, N_A,
                    [_x_spec(), _h_spec()], 1)(emb_author, hs_w)

    agg_c, tok = agg_paper(agg_c_k, xn_c, cs_a, cd_a, xn_c)
    agg_w, tok = agg_paper(agg_w_k, xn_w, ws_a, wd_a, tok)
    agg_r, tok = agg_author(agg_r_k, xn_r, rs_a, rd_a, tok)

    post2s = _tc_call(_post2s_body, N_P,
                      [_x_spec(), _x_spec(), _w_spec(), _w_spec(), _h_spec(),
                       _h_spec(), _h_spec(), _h_spec(), _b_spec()], 2)
    hn_c, hn_r = post2s(agg_c, agg_w, W1_cites, W1_writes, hd_c, hd_w,
                        hs_c, hs_r, (b1_cites + b1_writes).reshape(1, D))
    post1s = _tc_call(_post1s_body, N_A,
                      [_x_spec(), _w_spec(), _h_spec(), _h_spec(), _b_spec()], 1)
    hn_w = post1s(agg_r, W1_rev, hd_r, hs_w, b1_rev.reshape(1, D))

    agg2_c, tok = agg_paper(agg_c_k, hn_c, cs_a, cd_a, tok)
    agg2_w, tok = agg_paper(agg_w_k, hn_w, ws_a, wd_a, tok)
    agg2_r, tok = agg_author(agg_r_k, hn_r, rs_a, rd_a, tok)

    post2f = _tc_call(_post2f_body, N_P,
                      [_x_spec(), _x_spec(), _w_spec(), _w_spec(), _h_spec(),
                       _h_spec(), _b_spec()], 1)
    out_paper = post2f(agg2_c, agg2_w, W2_cites, W2_writes, hd_c, hd_w,
                       (b2_cites + b2_writes).reshape(1, D))
    post1f = _tc_call(_post1f_body, N_A,
                      [_x_spec(), _w_spec(), _h_spec(), _b_spec()], 1)
    out_author = post1f(agg2_r, W2_rev, hd_r, b2_rev.reshape(1, D))

    return (out_paper, out_author)


# speedup vs baseline: 1.0612x; 1.0466x over previous
import functools

import jax
import jax.numpy as jnp
from jax import lax
from jax.experimental import pallas as pl
from jax.experimental.pallas import tpu as pltpu
import jax.experimental.pallas.tpu_sc as plsc

N_P = 50000
N_A = 20000
D = 128
NC = 2
NS = 16
LANES = 16
B = 64
BD = 128
GR = NS * BD

CH_P = 12544
ACC_P = 12800
NPASS_P = 2
CH_A = 10000
ACC_A = 10368
NPASS_A = 1


def _round_up(x, m):
    return -(-x // m) * m

_MESH = plsc.VectorSubcoreMesh(core_axis_name="c", subcore_axis_name="s",
                               num_cores=NC, num_subcores=NS)


def _chunks(total, step):
    out = []
    r = 0
    while r < total:
        out.append((r, min(step, total - r)))
        r += step
    return out


def _make_agg(nb, ch, acc_rows, npass):
    et = nb * B
    out_rows = NC * npass * acc_rows
    rpt = acc_rows // NS
    zw = _chunks(rpt, 16)

    def body(x_hbm, src_hbm, dst_hbm, tok_hbm, out_hbm,
             src0, src1, src2, dst0, dst1, dst2, dl0, dl1, dl2,
             rows0, rows1, rows2,
             zbuf, acc, gs0, gs1, gs2, is0, is1, is2):
        c = lax.axis_index("c")
        s = lax.axis_index("s")
        iota = lax.iota(jnp.int32, LANES)
        srcs = (src0, src1, src2)
        dsts = (dst0, dst1, dst2)
        dls = (dl0, dl1, dl2)
        rows = (rows0, rows1, rows2)
        gss = (gs0, gs1, gs2)
        iss = (is0, is1, is2)

        def zb_fill(i, _):
            for j in range(D // LANES):
                zbuf[i, pl.ds(j * LANES, LANES)] = jnp.zeros((LANES,), jnp.float32)
            return 0
        lax.fori_loop(0, 16, zb_fill, 0)

        trash = ch + s * LANES + iota
        base = s * et

        def fetch_idx(buf, i):
            off = base + i * B
            pltpu.async_copy(src_hbm.at[pl.ds(off, B)], srcs[buf], iss[buf])
            pltpu.async_copy(dst_hbm.at[pl.ds(off, B)], dsts[buf], iss[buf])

        def wait_idx(buf):
            pltpu.make_async_copy(src_hbm.at[pl.ds(0, B)], srcs[buf], iss[buf]).wait()
            pltpu.make_async_copy(dst_hbm.at[pl.ds(0, B)], dsts[buf], iss[buf]).wait()

        def issue_gather(buf):
            pltpu.async_copy(x_hbm.at[srcs[buf]], rows[buf], gss[buf])

        def wait_gather(buf):
            pltpu.make_async_copy(x_hbm.at[srcs[buf]], rows[buf], gss[buf]).wait()

        def compute_dloc(buf, lo, hi):
            for j in range(B // LANES):
                dvec = dsts[buf][pl.ds(j * LANES, LANES)]
                inr = (dvec >= lo) & (dvec < hi)
                dls[buf][pl.ds(j * LANES, LANES)] = jnp.where(inr, dvec - lo, trash)

        def scatter(buf):
            pltpu.sync_copy(rows[buf], acc.at[dls[buf]], add=True)

        for p in range(npass):
            chunk = c * npass + p
            lo = chunk * ch
            hi = lo + ch

            for (r0, rn) in zw:
                pltpu.sync_copy(zbuf.at[pl.ds(0, rn)],
                                acc.at[pl.ds(s * rpt + r0, rn)])
            plsc.subcore_barrier()

            pltpu.sync_copy(src_hbm.at[pl.ds(base, B)], src0)
            pltpu.sync_copy(dst_hbm.at[pl.ds(base, B)], dst0)
            compute_dloc(0, lo, hi)
            issue_gather(0)
            pltpu.sync_copy(src_hbm.at[pl.ds(base + B, B)], src1)
            pltpu.sync_copy(dst_hbm.at[pl.ds(base + B, B)], dst1)
            compute_dloc(1, lo, hi)
            issue_gather(1)
            fetch_idx(2, 2)

            def grp(g, _):
                for sub in range(3):
                    i = 3 * g + sub
                    cur = sub
                    nx2 = (sub + 2) % 3
                    wait_idx(nx2)
                    compute_dloc(nx2, lo, hi)
                    issue_gather(nx2)
                    wait_gather(cur)

                    @pl.when(i + 3 < nb)
                    def _():
                        fetch_idx(cur, i + 3)
                    scatter(cur)
                return 0
            lax.fori_loop(0, (nb - 2) // 3, grp, 0)

            wait_gather(0)
            scatter(0)
            wait_gather(1)
            scatter(1)
            plsc.subcore_barrier()

            pltpu.sync_copy(acc.at[pl.ds(s * rpt, rpt)],
                            out_hbm.at[pl.ds(chunk * acc_rows + s * rpt, rpt)])
            plsc.subcore_barrier()

    kern = pl.kernel(
        body,
        out_type=jax.ShapeDtypeStruct((out_rows, D), jnp.float32),
        mesh=_MESH,
        scratch_types=(
            [pltpu.VMEM((B,), jnp.int32)] * 9
            + [pltpu.VMEM((B, D), jnp.float32)] * 3
            + [pltpu.VMEM((16, D), jnp.float32),
               pltpu.VMEM_SHARED((acc_rows, D), jnp.float32)]
            + [pltpu.SemaphoreType.DMA] * 6
        ),
        compiler_params=pltpu.CompilerParams(use_tc_tiling_on_sc=False),
    )
    return kern


def _unpad_agg(out, npass, acc_rows, ch, n):
    return out.reshape(NC * npass, acc_rows, D)[:, :ch].reshape(-1, D)[:n]


def _make_deg(e_pads, n_nodes, owners):
    n_arr = len(e_pads)
    n_hist = [_round_up(n + LANES, 8 * NS) for n in n_nodes]
    acc_rows = max(n_hist)

    def body(*refs):
        ins = refs[:n_arr]
        outs = refs[n_arr:2 * n_arr]
        idx0, idx1, onesb, zb, acc, is0, is1 = refs[2 * n_arr:]
        c = lax.axis_index("c")
        s = lax.axis_index("s")
        idxs = (idx0, idx1)
        iss = (is0, is1)

        def zb_fill(i, _):
            zb[i, pl.ds(0, LANES)] = jnp.zeros((LANES,), jnp.float32)
            return 0
        lax.fori_loop(0, 1024, zb_fill, 0)

        def ob_fill(i, _):
            onesb[i, pl.ds(0, LANES)] = jnp.ones((LANES,), jnp.float32)
            return 0
        lax.fori_loop(0, BD, ob_fill, 0)

        for a in range(n_arr):
            nh = n_hist[a]
            et = e_pads[a] // NS
            nb = et // BD
            rpt = nh // NS
            idx_hbm = ins[a]
            out_hbm = outs[a]
            base = s * et

            @pl.when(c == owners[a])
            def _():
                for (r0, rn) in _chunks(rpt, 1024):
                    pltpu.sync_copy(zb.at[pl.ds(0, rn)],
                                    acc.at[pl.ds(s * rpt + r0, rn)])
                plsc.subcore_barrier()

                pltpu.sync_copy(idx_hbm.at[pl.ds(base, BD)], idx0)
                pltpu.async_copy(idx_hbm.at[pl.ds(base + BD, BD)], idx1, is1)

                def cnt(g, _):
                    for sub in range(2):
                        i = 2 * g + sub
                        cur = sub
                        nxt = 1 - sub
                        pltpu.sync_copy(onesb, acc.at[idxs[cur]], add=True)

                        @pl.when(i + 2 < nb)
                        def _():
                            pltpu.async_copy(
                                idx_hbm.at[pl.ds(base + (i + 2) * BD, BD)],
                                idxs[cur], iss[cur])
                        pltpu.make_async_copy(
                            idx_hbm.at[pl.ds(0, BD)], idxs[nxt], iss[nxt]).wait()
                    return 0
                lax.fori_loop(0, (nb - 1) // 2, cnt, 0)
                pltpu.sync_copy(onesb, acc.at[idxs[0]], add=True)
                plsc.subcore_barrier()

                pltpu.sync_copy(acc.at[pl.ds(s * rpt, rpt)],
                                out_hbm.at[pl.ds(s * rpt, rpt)])
                plsc.subcore_barrier()

    kern = pl.kernel(
        body,
        out_type=[jax.ShapeDtypeStruct((n, LANES), jnp.float32)
                  for n in n_hist],
        mesh=_MESH,
        scratch_types=[
            pltpu.VMEM((BD,), jnp.int32),
            pltpu.VMEM((BD,), jnp.int32),
            pltpu.VMEM((BD, LANES), jnp.float32),
            pltpu.VMEM((1024, LANES), jnp.float32),
            pltpu.VMEM_SHARED((acc_rows, LANES), jnp.float32),
            pltpu.SemaphoreType.DMA,
            pltpu.SemaphoreType.DMA,
        ],
        compiler_params=pltpu.CompilerParams(use_tc_tiling_on_sc=False),
    )
    return kern


BR = 2000


def _rs(h):
    return lax.rsqrt(jnp.maximum(h[:, 0:1], 1.0))


def _scale2_body(x_ref, h1_ref, h2_ref, o1_ref, o2_ref):
    x = x_ref[...]
    o1_ref[...] = x * _rs(h1_ref[...])
    o2_ref[...] = x * _rs(h2_ref[...])


def _scale1_body(x_ref, h1_ref, o1_ref):
    o1_ref[...] = x_ref[...] * _rs(h1_ref[...])


def _post2s_body(a1_ref, a2_ref, w1_ref, w2_ref, hd1_ref, hd2_ref,
                 hs1_ref, hs2_ref, b_ref, o1_ref, o2_ref):
    h = (_rs(hd1_ref[...]) * jnp.dot(a1_ref[...], w1_ref[...],
                                     preferred_element_type=jnp.float32)
         + _rs(hd2_ref[...]) * jnp.dot(a2_ref[...], w2_ref[...],
                                       preferred_element_type=jnp.float32)
         + b_ref[...])
    o1_ref[...] = h * _rs(hs1_ref[...])
    o2_ref[...] = h * _rs(hs2_ref[...])


def _post2f_body(a1_ref, a2_ref, w1_ref, w2_ref, hd1_ref, hd2_ref, b_ref, o_ref):
    o_ref[...] = (_rs(hd1_ref[...]) * jnp.dot(a1_ref[...], w1_ref[...],
                                              preferred_element_type=jnp.float32)
                  + _rs(hd2_ref[...]) * jnp.dot(a2_ref[...], w2_ref[...],
                                                preferred_element_type=jnp.float32)
                  + b_ref[...])


def _post1s_body(a1_ref, w1_ref, hd1_ref, hs1_ref, b_ref, o_ref):
    h = (_rs(hd1_ref[...]) * jnp.dot(a1_ref[...], w1_ref[...],
                                     preferred_element_type=jnp.float32)
         + b_ref[...])
    o_ref[...] = h * _rs(hs1_ref[...])


def _post1f_body(a1_ref, w1_ref, hd1_ref, b_ref, o_ref):
    o_ref[...] = (_rs(hd1_ref[...]) * jnp.dot(a1_ref[...], w1_ref[...],
                                              preferred_element_type=jnp.float32)
                  + b_ref[...])


def _x_spec():
    return pl.BlockSpec((BR, D), lambda i: (i, 0))


def _h_spec():
    return pl.BlockSpec((BR, 1), lambda i: (i, 0))


def _w_spec():
    return pl.BlockSpec((D, D), lambda i: (0, 0))


def _b_spec():
    return pl.BlockSpec((1, D), lambda i: (0, 0))


def _tc_call(body, n, in_specs, n_out):
    out_spec = _x_spec()
    shape = jax.ShapeDtypeStruct((n, D), jnp.float32)
    if n_out == 1:
        return pl.pallas_call(body, grid=(n // BR,), in_specs=in_specs,
                              out_specs=out_spec, out_shape=shape)
    return pl.pallas_call(body, grid=(n // BR,), in_specs=in_specs,
                          out_specs=[out_spec] * n_out,
                          out_shape=[shape] * n_out)


def _pad_len(e):
    nb = -(-e // GR)
    return (nb + 1 - (nb % 2)) * GR


def _agg_nb(e):
    nb = -(-e // (NS * B))
    return nb + (2 - nb % 3) % 3


def _pad_agg_edges(src, dst, n_src):
    e = src.shape[0]
    p = _agg_nb(e) * NS * B - e
    ar = jnp.arange(p, dtype=jnp.int32)
    src_p = jnp.concatenate([src, (ar * 911) % n_src])
    dst_p = jnp.concatenate([dst, jnp.full((p,), -1, jnp.int32)])
    return src_p, dst_p


def _pad_deg_edges(idx, n):
    e = idx.shape[0]
    p = _pad_len(e) - e
    ar = jnp.arange(p, dtype=jnp.int32)
    return jnp.concatenate([idx, n + (ar % LANES)])


def kernel(xs, cites_src, cites_dst, writes_src, writes_dst, rev_writes_src,
           rev_writes_dst, emb_author, W1_cites, b1_cites, W1_writes, b1_writes,
           W1_rev, b1_rev, W2_cites, b2_cites, W2_writes, b2_writes, W2_rev,
           b2_rev):
    ec = _pad_len(cites_src.shape[0])
    ew = _pad_len(writes_src.shape[0])
    er = _pad_len(rev_writes_src.shape[0])
    nb_c = _agg_nb(cites_src.shape[0])
    nb_w = _agg_nb(writes_src.shape[0])
    nb_r = _agg_nb(rev_writes_src.shape[0])

    cs_a, cd_a = _pad_agg_edges(cites_src, cites_dst, N_P)
    ws_a, wd_a = _pad_agg_edges(writes_src, writes_dst, N_A)
    rs_a, rd_a = _pad_agg_edges(rev_writes_src, rev_writes_dst, N_P)

    deg_in = [_pad_deg_edges(cites_src, N_P), _pad_deg_edges(cites_dst, N_P),
              _pad_deg_edges(writes_src, N_A), _pad_deg_edges(writes_dst, N_P),
              _pad_deg_edges(rev_writes_src, N_P), _pad_deg_edges(rev_writes_dst, N_A)]

    deg_kern = _make_deg([ec, ec, ew, ew, er, er],
                         [N_P, N_P, N_A, N_P, N_P, N_A],
                         [0, 0, 1, 1, 1, 1])
    hists = deg_kern(*deg_in)
    hs_c = hists[0][:N_P, 0:1]
    hd_c = hists[1][:N_P, 0:1]
    hs_w = hists[2][:N_A, 0:1]
    hd_w = hists[3][:N_P, 0:1]
    hs_r = hists[4][:N_P, 0:1]
    hd_r = hists[5][:N_A, 0:1]

    agg_c_k = _make_agg(nb_c, CH_P, ACC_P, NPASS_P)
    agg_w_k = _make_agg(nb_w, CH_P, ACC_P, NPASS_P)
    agg_r_k = _make_agg(nb_r, CH_A, ACC_A, NPASS_A)

    def agg_paper(kern, x, s, d, tok):
        raw = kern(x, s, d, tok[:8, :D])
        return _unpad_agg(raw, NPASS_P, ACC_P, CH_P, N_P), raw

    def agg_author(kern, x, s, d, tok):
        raw = kern(x, s, d, tok[:8, :D])
        return _unpad_agg(raw, NPASS_A, ACC_A, CH_A, N_A), raw

    xn_c, xn_r = _tc_call(_scale2_body, N_P,
                          [_x_spec(), _h_spec(), _h_spec()], 2)(xs, hs_c, hs_r)
    xn_w = _tc_call(_scale1_body, N_A,
                    [_x_spec(), _h_spec()], 1)(emb_author, hs_w)

    agg_c, tok = agg_paper(agg_c_k, xn_c, cs_a, cd_a, xn_c)
    agg_w, tok = agg_paper(agg_w_k, xn_w, ws_a, wd_a, tok)
    agg_r, tok = agg_author(agg_r_k, xn_r, rs_a, rd_a, tok)

    post2s = _tc_call(_post2s_body, N_P,
                      [_x_spec(), _x_spec(), _w_spec(), _w_spec(), _h_spec(),
                       _h_spec(), _h_spec(), _h_spec(), _b_spec()], 2)
    hn_c, hn_r = post2s(agg_c, agg_w, W1_cites, W1_writes, hd_c, hd_w,
                        hs_c, hs_r, (b1_cites + b1_writes).reshape(1, D))
    post1s = _tc_call(_post1s_body, N_A,
                      [_x_spec(), _w_spec(), _h_spec(), _h_spec(), _b_spec()], 1)
    hn_w = post1s(agg_r, W1_rev, hd_r, hs_w, b1_rev.reshape(1, D))

    agg2_c, tok = agg_paper(agg_c_k, hn_c, cs_a, cd_a, tok)
    agg2_w, tok = agg_paper(agg_w_k, hn_w, ws_a, wd_a, tok)
    agg2_r, tok = agg_author(agg_r_k, hn_r, rs_a, rd_a, tok)

    post2f = _tc_call(_post2f_body, N_P,
                      [_x_spec(), _x_spec(), _w_spec(), _w_spec(), _h_spec(),
                       _h_spec(), _b_spec()], 1)
    out_paper = post2f(agg2_c, agg2_w, W2_cites, W2_writes, hd_c, hd_w,
                       (b2_cites + b2_writes).reshape(1, D))
    post1f = _tc_call(_post1f_body, N_A,
                      [_x_spec(), _w_spec(), _h_spec(), _b_spec()], 1)
    out_author = post1f(agg2_r, W2_rev, hd_r, b2_rev.reshape(1, D))

    return (out_paper, out_author)
